# baseline (speedup 1.0000x reference)
import functools

import jax
import jax.numpy as jnp
from jax import lax
from jax.experimental import pallas as pl
from jax.experimental.pallas import tpu as pltpu
from jax.experimental.pallas import tpu_sc as plsc

_E = 320000
_N = 10000
_D = 128
_NSPLIT = 4
_EPART = _E // _NSPLIT
_EBLK = 16000
_NCORES = 2
_NSUB = 16
_CK = 128
_EPT = _EPART // _NSUB
_NCH = _EPT // _CK
_TAIL = _EPT - _NCH * _CK
_ZROWS = 125
_NBLK = 2000


def _eye(n):
    return (lax.broadcasted_iota(jnp.int32, (n, n), 0) ==
            lax.broadcasted_iota(jnp.int32, (n, n), 1)).astype(jnp.float32)


_TDN = (((0,), (0,)), ((), ()))


def _edge_body(vt_ref, rt_ref, w1_ref, b1_ref, out_ref):
    f32 = jnp.float32
    zT = lax.dot_general(w1_ref[...], rt_ref[...], _TDN,
                         preferred_element_type=f32) + b1_ref[...]
    aT = zT * jax.nn.sigmoid(zT)
    vT = vt_ref[...]
    invT = lax.rsqrt(jnp.sum(vT * vT, axis=0, keepdims=True) + 1e-9)
    pt0 = jnp.concatenate([vT[0:1, :] * invT * aT,
                           vT[1:2, :] * invT * aT], axis=0)
    pt1 = jnp.concatenate([vT[2:3, :] * invT * aT, aT], axis=0)
    out_ref[0] = lax.dot_general(pt0, _eye(128), _TDN,
                                 preferred_element_type=f32)
    out_ref[1] = lax.dot_general(pt1, _eye(128), _TDN,
                                 preferred_element_type=f32)


def _edge_stage(vectorsT, radialT, W_r1, b_r1c, part):
    grid = _EPART // _EBLK
    off = part * grid
    return pl.pallas_call(
        _edge_body,
        grid=(grid,),
        in_specs=[
            pl.BlockSpec((3, _EBLK), lambda i: (0, i + off)),
            pl.BlockSpec((8, _EBLK), lambda i: (0, i + off)),
            pl.BlockSpec((8, 64), lambda i: (0, 0)),
            pl.BlockSpec((64, 1), lambda i: (0, 0)),
        ],
        out_specs=pl.BlockSpec((2, _EBLK, 128), lambda i: (0, i, 0)),
        out_shape=jax.ShapeDtypeStruct((2, _EPART, 128), jnp.float32),
    )(vectorsT, radialT, W_r1, b_r1c)


def _sc_body(p_hbm, recv_hbm, zeros_hbm, out_hbm,
             pb0, pb1, ib0, ib1, tibuf, acc, sem0, sem1):
    c = lax.axis_index("c")
    s = lax.axis_index("s")
    pltpu.sync_copy(zeros_hbm, pb0.at[pl.ds(0, _ZROWS)])
    for k in range(5):
        pltpu.sync_copy(pb0.at[pl.ds(0, _ZROWS)],
                        acc.at[pl.ds(s * 625 + k * _ZROWS, _ZROWS)])
    plsc.subcore_barrier()

    base = s * _EPT

    def issue_load(k, ib, pb, sem):
        st = base + k * _CK
        pltpu.async_copy(recv_hbm.at[pl.ds(st, _CK)], ib, sem)
        pltpu.async_copy(p_hbm.at[c, pl.ds(st, _CK)], pb, sem)

    def wait_load(ib, pb, sem):
        pltpu.make_async_copy(recv_hbm.at[pl.ds(0, _CK)], ib, sem).wait()
        pltpu.make_async_copy(p_hbm.at[0, pl.ds(0, _CK)], pb, sem).wait()

    issue_load(0, ib0, pb0, sem0)
    issue_load(1, ib1, pb1, sem1)
    npair = _NCH // 2

    def pair(j, carry):
        wait_load(ib0, pb0, sem0)
        pltpu.sync_copy(pb0, acc.at[ib0], add=True)
        issue_load(2 * j + 2, ib0, pb0, sem0)
        wait_load(ib1, pb1, sem1)
        pltpu.sync_copy(pb1, acc.at[ib1], add=True)

        @pl.when(j < npair - 1)
        def _():
            issue_load(2 * j + 3, ib1, pb1, sem1)
        return carry

    lax.fori_loop(0, npair, pair, 0)
    wait_load(ib0, pb0, sem0)
    pltpu.sync_copy(pb0, acc.at[ib0], add=True)
    st = base + _NCH * _CK
    pltpu.sync_copy(recv_hbm.at[pl.ds(st, _TAIL)], tibuf)
    pltpu.sync_copy(p_hbm.at[c, pl.ds(st, _TAIL)], pb0.at[pl.ds(0, _TAIL)])
    pltpu.sync_copy(pb0.at[pl.ds(0, _TAIL)], acc.at[tibuf], add=True)
    plsc.subcore_barrier()

    @pl.when(s < 10)
    def _drain():
        pltpu.sync_copy(acc.at[pl.ds(s * 1000, 1000)],
                        out_hbm.at[c, pl.ds(s * 1000, 1000)])


def _scatter_stage(P, recv32, zeros):
    mesh = plsc.VectorSubcoreMesh(core_axis_name="c", subcore_axis_name="s",
                                  num_cores=_NCORES, num_subcores=_NSUB)
    kern = pl.kernel(
        _sc_body,
        out_type=jax.ShapeDtypeStruct((2, _N, 128), jnp.float32),
        mesh=mesh,
        scratch_types=[
            pltpu.VMEM((_CK, 128), jnp.float32),
            pltpu.VMEM((_CK, 128), jnp.float32),
            pltpu.VMEM((_CK,), jnp.int32),
            pltpu.VMEM((_CK,), jnp.int32),
            pltpu.VMEM((_TAIL,), jnp.int32),
            pltpu.VMEM_SHARED((_N, 128), jnp.float32),
            pltpu.SemaphoreType.DMA,
            pltpu.SemaphoreType.DMA,
        ],
        compiler_params=pltpu.CompilerParams(use_tc_tiling_on_sc=True),
    )
    return kern(P, recv32, zeros)


def _node_body(ba_ref, bb_ref, bc_ref, bd_ref, nf_ref, ns_ref, se_ref,
               avg_ref, wr2_ref, wm_ref, wsc_ref, wse_ref, g1_ref, wad_ref,
               g2_ref, b2_ref, wro_ref, x_ref, ro_ref):
    f32 = jnp.float32
    scale = 1.0 / avg_ref[...]
    wr2 = wr2_ref[...]
    c00 = jnp.dot(wr2, wm_ref[0], preferred_element_type=f32)
    c01 = jnp.dot(wr2, wm_ref[1], preferred_element_type=f32)
    c10 = jnp.dot(wr2, wm_ref[2], preferred_element_type=f32)
    c11 = jnp.dot(wr2, wm_ref[3], preferred_element_type=f32)
    C0 = jnp.concatenate([c00, c01], axis=0) * scale
    C1 = jnp.concatenate([c10, c11], axis=0) * scale
    x = (jnp.dot(ba_ref[0] + bb_ref[0] + bc_ref[0] + bd_ref[0], C0,
                 preferred_element_type=f32)
         + jnp.dot(ba_ref[1] + bb_ref[1] + bc_ref[1] + bd_ref[1], C1,
                   preferred_element_type=f32))
    nf = nf_ref[...]
    G = jnp.dot(se_ref[...], wse_ref[...], preferred_element_type=f32)
    oh = (lax.broadcasted_iota(jnp.int32, (_NBLK, 10), 1)
          == ns_ref[...]).astype(f32)
    gate = jax.nn.sigmoid(jnp.dot(oh, G, preferred_element_type=f32))
    x = x + jnp.dot(nf, wsc_ref[...], preferred_element_type=f32) * gate
    mu = jnp.mean(x, axis=1, keepdims=True)
    var = jnp.mean(x * x, axis=1, keepdims=True) - mu * mu
    x = (x - mu) * lax.rsqrt(var + 1e-6) * g1_ref[...]
    x = x + jnp.dot(nf, wad_ref[...], preferred_element_type=f32)
    mu2 = jnp.mean(x, axis=1, keepdims=True)
    var2 = jnp.mean(x * x, axis=1, keepdims=True) - mu2 * mu2
    x = (x - mu2) * lax.rsqrt(var2 + 1e-6) * g2_ref[...] + b2_ref[...]
    x_ref[...] = x
    ro_ref[...] = jnp.dot(x, wro_ref[...], preferred_element_type=f32)


def _node_stage(Bs, node_feats, ns_col, species_embed, avg, W_r2, Wm,
                W_sc, W_se, gamma, W_adapt, gamma2, beta2, W_ro):
    bspec = pl.BlockSpec((2, _NBLK, 128), lambda i: (0, i, 0))
    full = pl.BlockSpec
    return pl.pallas_call(
        _node_body,
        grid=(_N // _NBLK,),
        in_specs=[bspec, bspec, bspec, bspec,
                  pl.BlockSpec((_NBLK, 128), lambda i: (i, 0)),
                  pl.BlockSpec((_NBLK, 1), lambda i: (i, 0)),
                  full((10, 64), lambda i: (0, 0)),
                  full((1, 1), lambda i: (0, 0)),
                  full((64, 128), lambda i: (0, 0)),
                  full((4, 128, 128), lambda i: (0, 0, 0)),
                  full((128, 128), lambda i: (0, 0)),
                  full((64, 128), lambda i: (0, 0)),
                  full((1, 128), lambda i: (0, 0)),
                  full((128, 128), lambda i: (0, 0)),
                  full((1, 128), lambda i: (0, 0)),
                  full((1, 128), lambda i: (0, 0)),
                  full((128, 16), lambda i: (0, 0))],
        out_specs=(pl.BlockSpec((_NBLK, 128), lambda i: (i, 0)),
                   pl.BlockSpec((_NBLK, 16), lambda i: (i, 0))),
        out_shape=(jax.ShapeDtypeStruct((_N, _D), jnp.float32),
                   jax.ShapeDtypeStruct((_N, 16), jnp.float32)),
    )(*Bs, node_feats, ns_col, species_embed, avg, W_r2, Wm,
      W_sc, W_se, gamma, W_adapt, gamma2, beta2, W_ro)


def kernel(vectors, node_feats, node_species, radial_embedding, receivers,
           species_embed, avg_num_neighbors, W_r1, b_r1, W_r2, W_msg,
           W_sc, W_se, gamma, W_adapt, gamma2, beta2, W_ro):
    vT = vectors.T
    rT = radial_embedding.T
    b1c = b_r1.reshape(64, 1)
    recv32 = receivers.astype(jnp.int32)
    zeros = jnp.zeros((_ZROWS, 128), jnp.float32)
    Bs = []
    for q in range(_NSPLIT):
        Pq = _edge_stage(vT, rT, W_r1, b1c, q)
        Bs.append(_scatter_stage(
            Pq, lax.slice_in_dim(recv32, q * _EPART, (q + 1) * _EPART), zeros))
    Wm = W_msg.reshape(_D, 4, _D).transpose(1, 0, 2)
    x, ro = _node_stage(
        Bs, node_feats, node_species.astype(jnp.int32).reshape(_N, 1),
        species_embed, avg_num_neighbors.reshape(1, 1).astype(jnp.float32),
        W_r2, Wm, W_sc, W_se, gamma.reshape(1, _D), W_adapt,
        gamma2.reshape(1, _D), beta2.reshape(1, _D), W_ro)
    return (x, ro)


# speedup vs baseline: 1.0955x; 1.0955x over previous
import functools

import jax
import jax.numpy as jnp
from jax import lax
from jax.experimental import pallas as pl
from jax.experimental.pallas import tpu as pltpu
from jax.experimental.pallas import tpu_sc as plsc

_E = 320000
_N = 10000
_D = 128
_NSPLIT = 2
_EPART = _E // _NSPLIT
_EBLK = 16000
_NCORES = 2
_NSUB = 16
_CK = 128
_EPT = _EPART // _NSUB
_NCH = _EPT // _CK
_TAIL = _EPT - _NCH * _CK
_ZROWS = 125
_NBLK = 2000


def _eye(n):
    return (lax.broadcasted_iota(jnp.int32, (n, n), 0) ==
            lax.broadcasted_iota(jnp.int32, (n, n), 1)).astype(jnp.float32)


_TDN = (((0,), (0,)), ((), ()))


def _edge_body(vt_ref, rt_ref, w1_ref, b1_ref, out_ref):
    f32 = jnp.float32
    zT = lax.dot_general(w1_ref[...], rt_ref[...], _TDN,
                         preferred_element_type=f32) + b1_ref[...]
    aT = zT * jax.nn.sigmoid(zT)
    vT = vt_ref[...]
    invT = lax.rsqrt(jnp.sum(vT * vT, axis=0, keepdims=True) + 1e-9)
    pt0 = jnp.concatenate([vT[0:1, :] * invT * aT,
                           vT[1:2, :] * invT * aT], axis=0)
    pt1 = jnp.concatenate([vT[2:3, :] * invT * aT, aT], axis=0)
    out_ref[0] = lax.dot_general(pt0, _eye(128), _TDN,
                                 preferred_element_type=f32)
    out_ref[1] = lax.dot_general(pt1, _eye(128), _TDN,
                                 preferred_element_type=f32)


def _edge_stage(vectorsT, radialT, W_r1, b_r1c, part):
    grid = _EPART // _EBLK
    off = part * grid
    return pl.pallas_call(
        _edge_body,
        grid=(grid,),
        in_specs=[
            pl.BlockSpec((3, _EBLK), lambda i: (0, i + off)),
            pl.BlockSpec((8, _EBLK), lambda i: (0, i + off)),
            pl.BlockSpec((8, 64), lambda i: (0, 0)),
            pl.BlockSpec((64, 1), lambda i: (0, 0)),
        ],
        out_specs=pl.BlockSpec((2, _EBLK, 128), lambda i: (0, i, 0)),
        out_shape=jax.ShapeDtypeStruct((2, _EPART, 128), jnp.float32),
    )(vectorsT, radialT, W_r1, b_r1c)


def _sc_body(p_hbm, recv_hbm, zeros_hbm, out_hbm,
             pb0, pb1, ib0, ib1, tibuf, acc, sem0, sem1):
    c = lax.axis_index("c")
    s = lax.axis_index("s")
    pltpu.sync_copy(zeros_hbm, pb0.at[pl.ds(0, _ZROWS)])
    for k in range(5):
        pltpu.sync_copy(pb0.at[pl.ds(0, _ZROWS)],
                        acc.at[pl.ds(s * 625 + k * _ZROWS, _ZROWS)])
    plsc.subcore_barrier()

    base = s * _EPT

    def issue_load(k, ib, pb, sem):
        st = base + k * _CK
        pltpu.async_copy(recv_hbm.at[pl.ds(st, _CK)], ib, sem)
        pltpu.async_copy(p_hbm.at[c, pl.ds(st, _CK)], pb, sem)

    def wait_load(ib, pb, sem):
        pltpu.make_async_copy(recv_hbm.at[pl.ds(0, _CK)], ib, sem).wait()
        pltpu.make_async_copy(p_hbm.at[0, pl.ds(0, _CK)], pb, sem).wait()

    issue_load(0, ib0, pb0, sem0)
    issue_load(1, ib1, pb1, sem1)
    npair = _NCH // 2
    odd = _NCH % 2 == 1

    def pair(j, carry):
        wait_load(ib0, pb0, sem0)
        pltpu.sync_copy(pb0, acc.at[ib0], add=True)
        if odd:
            issue_load(2 * j + 2, ib0, pb0, sem0)
        else:
            @pl.when(j < npair - 1)
            def _():
                issue_load(2 * j + 2, ib0, pb0, sem0)
        wait_load(ib1, pb1, sem1)
        pltpu.sync_copy(pb1, acc.at[ib1], add=True)

        @pl.when(j < npair - 1)
        def _():
            issue_load(2 * j + 3, ib1, pb1, sem1)
        return carry

    lax.fori_loop(0, npair, pair, 0)
    if odd:
        wait_load(ib0, pb0, sem0)
        pltpu.sync_copy(pb0, acc.at[ib0], add=True)
    st = base + _NCH * _CK
    pltpu.sync_copy(recv_hbm.at[pl.ds(st, _TAIL)], tibuf)
    pltpu.sync_copy(p_hbm.at[c, pl.ds(st, _TAIL)], pb0.at[pl.ds(0, _TAIL)])
    pltpu.sync_copy(pb0.at[pl.ds(0, _TAIL)], acc.at[tibuf], add=True)
    plsc.subcore_barrier()

    @pl.when(s < 10)
    def _drain():
        pltpu.sync_copy(acc.at[pl.ds(s * 1000, 1000)],
                        out_hbm.at[c, pl.ds(s * 1000, 1000)])


def _scatter_stage(P, recv32, zeros):
    mesh = plsc.VectorSubcoreMesh(core_axis_name="c", subcore_axis_name="s",
                                  num_cores=_NCORES, num_subcores=_NSUB)
    kern = pl.kernel(
        _sc_body,
        out_type=jax.ShapeDtypeStruct((2, _N, 128), jnp.float32),
        mesh=mesh,
        scratch_types=[
            pltpu.VMEM((_CK, 128), jnp.float32),
            pltpu.VMEM((_CK, 128), jnp.float32),
            pltpu.VMEM((_CK,), jnp.int32),
            pltpu.VMEM((_CK,), jnp.int32),
            pltpu.VMEM((_TAIL,), jnp.int32),
            pltpu.VMEM_SHARED((_N, 128), jnp.float32),
            pltpu.SemaphoreType.DMA,
            pltpu.SemaphoreType.DMA,
        ],
        compiler_params=pltpu.CompilerParams(use_tc_tiling_on_sc=True),
    )
    return kern(P, recv32, zeros)


def _node_body(*refs):
    b_refs = refs[:_NSPLIT]
    (nf_ref, ns_ref, se_ref, avg_ref, wr2_ref, wm_ref, wsc_ref, wse_ref,
     g1_ref, wad_ref, g2_ref, b2_ref, wro_ref, x_ref, ro_ref) = refs[_NSPLIT:]
    f32 = jnp.float32
    scale = 1.0 / avg_ref[...]
    wr2 = wr2_ref[...]
    c00 = jnp.dot(wr2, wm_ref[0], preferred_element_type=f32)
    c01 = jnp.dot(wr2, wm_ref[1], preferred_element_type=f32)
    c10 = jnp.dot(wr2, wm_ref[2], preferred_element_type=f32)
    c11 = jnp.dot(wr2, wm_ref[3], preferred_element_type=f32)
    C0 = jnp.concatenate([c00, c01], axis=0) * scale
    C1 = jnp.concatenate([c10, c11], axis=0) * scale
    B0 = b_refs[0][0]
    B1 = b_refs[0][1]
    for br in b_refs[1:]:
        B0 = B0 + br[0]
        B1 = B1 + br[1]
    x = (jnp.dot(B0, C0, preferred_element_type=f32)
         + jnp.dot(B1, C1, preferred_element_type=f32))
    nf = nf_ref[...]
    G = jnp.dot(se_ref[...], wse_ref[...], preferred_element_type=f32)
    oh = (lax.broadcasted_iota(jnp.int32, (_NBLK, 10), 1)
          == ns_ref[...]).astype(f32)
    gate = jax.nn.sigmoid(jnp.dot(oh, G, preferred_element_type=f32))
    x = x + jnp.dot(nf, wsc_ref[...], preferred_element_type=f32) * gate
    mu = jnp.mean(x, axis=1, keepdims=True)
    var = jnp.mean(x * x, axis=1, keepdims=True) - mu * mu
    x = (x - mu) * lax.rsqrt(var + 1e-6) * g1_ref[...]
    x = x + jnp.dot(nf, wad_ref[...], preferred_element_type=f32)
    mu2 = jnp.mean(x, axis=1, keepdims=True)
    var2 = jnp.mean(x * x, axis=1, keepdims=True) - mu2 * mu2
    x = (x - mu2) * lax.rsqrt(var2 + 1e-6) * g2_ref[...] + b2_ref[...]
    x_ref[...] = x
    ro_ref[...] = jnp.dot(x, wro_ref[...], preferred_element_type=f32)


def _node_stage(Bs, node_feats, ns_col, species_embed, avg, W_r2, Wm,
                W_sc, W_se, gamma, W_adapt, gamma2, beta2, W_ro):
    bspec = pl.BlockSpec((2, _NBLK, 128), lambda i: (0, i, 0))
    full = pl.BlockSpec
    return pl.pallas_call(
        _node_body,
        grid=(_N // _NBLK,),
        in_specs=[bspec] * _NSPLIT + [
                  pl.BlockSpec((_NBLK, 128), lambda i: (i, 0)),
                  pl.BlockSpec((_NBLK, 1), lambda i: (i, 0)),
                  full((10, 64), lambda i: (0, 0)),
                  full((1, 1), lambda i: (0, 0)),
                  full((64, 128), lambda i: (0, 0)),
                  full((4, 128, 128), lambda i: (0, 0, 0)),
                  full((128, 128), lambda i: (0, 0)),
                  full((64, 128), lambda i: (0, 0)),
                  full((1, 128), lambda i: (0, 0)),
                  full((128, 128), lambda i: (0, 0)),
                  full((1, 128), lambda i: (0, 0)),
                  full((1, 128), lambda i: (0, 0)),
                  full((128, 16), lambda i: (0, 0))],
        out_specs=(pl.BlockSpec((_NBLK, 128), lambda i: (i, 0)),
                   pl.BlockSpec((_NBLK, 16), lambda i: (i, 0))),
        out_shape=(jax.ShapeDtypeStruct((_N, _D), jnp.float32),
                   jax.ShapeDtypeStruct((_N, 16), jnp.float32)),
    )(*Bs, node_feats, ns_col, species_embed, avg, W_r2, Wm,
      W_sc, W_se, gamma, W_adapt, gamma2, beta2, W_ro)


def kernel(vectors, node_feats, node_species, radial_embedding, receivers,
           species_embed, avg_num_neighbors, W_r1, b_r1, W_r2, W_msg,
           W_sc, W_se, gamma, W_adapt, gamma2, beta2, W_ro):
    vT = vectors.T
    rT = radial_embedding.T
    b1c = b_r1.reshape(64, 1)
    recv32 = receivers.astype(jnp.int32)
    zeros = jnp.zeros((_ZROWS, 128), jnp.float32)
    Bs = []
    for q in range(_NSPLIT):
        Pq = _edge_stage(vT, rT, W_r1, b1c, q)
        Bs.append(_scatter_stage(
            Pq, lax.slice_in_dim(recv32, q * _EPART, (q + 1) * _EPART), zeros))
    Wm = W_msg.reshape(_D, 4, _D).transpose(1, 0, 2)
    x, ro = _node_stage(
        Bs, node_feats, node_species.astype(jnp.int32).reshape(_N, 1),
        species_embed, avg_num_neighbors.reshape(1, 1).astype(jnp.float32),
        W_r2, Wm, W_sc, W_se, gamma.reshape(1, _D), W_adapt,
        gamma2.reshape(1, _D), beta2.reshape(1, _D), W_ro)
    return (x, ro)
